# Initial kernel scaffold; baseline (speedup 1.0000x reference)
import jax
import jax.numpy as jnp
from jax.experimental import pallas as pl


def kernel(node_features, pos_edge_index, neg_edge_index, W_self1, W_neigh1, b1, W_self2, W_neigh2, b2):
    raise NotImplementedError("write your pallas kernel here")


# speedup vs baseline: 31.1726x; 31.1726x over previous
import functools

import jax
import jax.numpy as jnp
from jax import lax
from jax.experimental import pallas as pl
from jax.experimental.pallas import tpu as pltpu
from jax.experimental.pallas import tpu_sc as plsc

NC = 2
NS = 16
NW = NC * NS
LANES = 16
DHALF = 8
DPAD = 16
ZCH = 272


def _mesh():
    return plsc.VectorSubcoreMesh(
        core_axis_name="c", subcore_axis_name="s", num_cores=NC, num_subcores=NS
    )


_SC_PARAMS = pltpu.CompilerParams(
    use_tc_tiling_on_sc=False, needs_layout_passes=False
)


def _make_agg(n_nodes, n_edges):
    epw = n_edges // NW
    ch = 2000
    nit = epw // ch
    rows_per_tile = n_nodes // NS
    nz = rows_per_tile // ZCH

    @functools.partial(
        pl.kernel,
        mesh=_mesh(),
        compiler_params=_SC_PARAMS,
        out_type=jax.ShapeDtypeStruct((NC, 2, n_nodes, DHALF), jnp.float32),
        scratch_types=[
            pltpu.VMEM((ch,), jnp.int32),
            pltpu.VMEM((ch,), jnp.int32),
            pltpu.VMEM((ch, DHALF), jnp.float32),
            pltpu.VMEM_SHARED((n_nodes, DHALF), jnp.float32),
            pltpu.SemaphoreType.DMA,
        ],
    )
    def agg(xa_hbm, xb_hbm, src_hbm, dst_hbm, zer_hbm, out_hbm,
            sidx, didx, rows, acc, sem):
        cid = lax.axis_index("c")
        sid = lax.axis_index("s")
        wid = sid * NC + cid
        row0 = sid * rows_per_tile
        base0 = wid * epw

        for ph, x_hbm in enumerate((xa_hbm, xb_hbm)):
            def zacc(i, _):
                pltpu.sync_copy(zer_hbm, acc.at[pl.ds(row0 + i * ZCH, ZCH)])
                return 0
            lax.fori_loop(0, nz, zacc, 0)
            plsc.subcore_barrier()

            def step(i, _):
                base = pl.multiple_of(base0 + i * ch, 8)
                pltpu.sync_copy(src_hbm.at[pl.ds(base, ch)], sidx)
                pltpu.sync_copy(dst_hbm.at[pl.ds(base, ch)], didx)
                pltpu.async_copy(x_hbm.at[sidx], rows, sem).wait()
                pltpu.sync_copy(rows, acc.at[didx], add=True)
                return 0
            lax.fori_loop(0, nit, step, 0)
            plsc.subcore_barrier()

            pltpu.sync_copy(
                acc.at[pl.ds(row0, rows_per_tile)],
                out_hbm.at[cid, ph, pl.ds(row0, rows_per_tile)],
            )
            plsc.subcore_barrier()

    return agg


def _make_combine(n_nodes, split_out):
    blk = n_nodes // 32
    grid = n_nodes // blk
    bspec8 = pl.BlockSpec((blk, DHALF), lambda i: (i, 0))
    wspec = pl.BlockSpec((DHALF, DPAD), lambda i: (0, 0))

    def body(xa_ref, xb_ref, a0a_ref, a1a_ref, a0b_ref, a1b_ref,
             wsa_ref, wsb_ref, wna_ref, wnb_ref, b_ref, *out_refs):
        comb_a = a0a_ref[...] + a1a_ref[...]
        comb_b = a0b_ref[...] + a1b_ref[...]
        deg = jnp.maximum(comb_b[:, 2:3], 1.0)
        dot = functools.partial(jnp.dot, preferred_element_type=jnp.float32)
        h = (
            dot(xa_ref[...], wsa_ref[...])
            + dot(xb_ref[...], wsb_ref[...])
            + dot(comb_a / deg, wna_ref[...])
            + dot(comb_b / deg, wnb_ref[...])
            + b_ref[0:1, :]
        )
        if split_out:
            out_refs[0][...] = h[:, :DHALF]
            out_refs[1][...] = h[:, DHALF:]
        else:
            out_refs[0][...] = h

    if split_out:
        out_shape = (
            jax.ShapeDtypeStruct((n_nodes, DHALF), jnp.float32),
            jax.ShapeDtypeStruct((n_nodes, DHALF), jnp.float32),
        )
        out_specs = (bspec8, bspec8)
    else:
        out_shape = jax.ShapeDtypeStruct((n_nodes, DPAD), jnp.float32)
        out_specs = pl.BlockSpec((blk, DPAD), lambda i: (i, 0))

    return pl.pallas_call(
        body,
        grid=(grid,),
        in_specs=[bspec8] * 6 + [wspec] * 4
        + [pl.BlockSpec((8, DPAD), lambda i: (0, 0))],
        out_specs=out_specs,
        out_shape=out_shape,
    )


def _make_score(n_nodes, n_edges, n_feat):
    epw = n_edges // NW
    ch = 2000
    nit = epw // ch
    ngrp = ch // LANES

    @functools.partial(
        pl.kernel,
        mesh=_mesh(),
        compiler_params=_SC_PARAMS,
        out_type=(
            jax.ShapeDtypeStruct((n_edges,), jnp.float32),
            jax.ShapeDtypeStruct((n_edges,), jnp.float32),
        ),
        scratch_types=[
            pltpu.VMEM((ch,), jnp.int32),
            pltpu.VMEM((ch,), jnp.int32),
            pltpu.VMEM((ch, DPAD), jnp.float32),
            pltpu.VMEM((ch, DPAD), jnp.float32),
            pltpu.VMEM((ch,), jnp.float32),
            pltpu.SemaphoreType.DMA,
            pltpu.SemaphoreType.DMA,
        ],
    )
    def score(h_hbm, pu_hbm, pv_hbm, nu_hbm, nv_hbm, pos_out, neg_out,
              uidx, vidx, urows, vrows, sc, semu, semv):
        cid = lax.axis_index("c")
        sid = lax.axis_index("s")
        wid = sid * NC + cid
        base0 = wid * epw
        lane = lax.iota(jnp.int32, 16)

        def run(u_hbm, v_hbm, out_hbm):
            def step(i, _):
                base = pl.multiple_of(base0 + i * ch, 8)
                pltpu.sync_copy(u_hbm.at[pl.ds(base, ch)], uidx)
                pltpu.sync_copy(v_hbm.at[pl.ds(base, ch)], vidx)
                cu = pltpu.async_copy(h_hbm.at[uidx], urows, semu)
                cv = pltpu.async_copy(h_hbm.at[vidx], vrows, semv)
                cu.wait()
                cv.wait()

                def grp(g, _):
                    evec = lane + g * LANES
                    acc = jnp.zeros((16,), jnp.float32)
                    for l in range(n_feat):
                        lvec = jnp.full((16,), l, jnp.int32)
                        gu = plsc.load_gather(urows, [evec, lvec])
                        gv = plsc.load_gather(vrows, [evec, lvec])
                        acc = acc + gu * gv
                    sc[pl.ds(pl.multiple_of(g * LANES, 8), LANES)] = acc
                    return 0
                lax.fori_loop(0, ngrp, grp, 0)
                pltpu.sync_copy(sc, out_hbm.at[pl.ds(base, ch)])
                return 0
            lax.fori_loop(0, nit, step, 0)

        run(pu_hbm, pv_hbm, pos_out)
        run(nu_hbm, nv_hbm, neg_out)

    return score


def kernel(node_features, pos_edge_index, neg_edge_index,
           W_self1, W_neigh1, b1, W_self2, W_neigh2, b2):
    n_nodes, d = node_features.shape
    n_edges = pos_edge_index.shape[1]
    n_pad = -(-n_nodes // 128) * 128

    xa = jnp.zeros((n_pad, DHALF), jnp.float32).at[:n_nodes].set(
        node_features[:, :DHALF])
    xb = jnp.zeros((n_pad, DHALF), jnp.float32)
    xb = xb.at[:n_nodes, :d - DHALF].set(node_features[:, DHALF:])
    xb = xb.at[:n_nodes, 2].set(1.0)

    def wsplit(w):
        w16 = jnp.zeros((d, DPAD), jnp.float32).at[:, :d].set(w)
        wa = w16[:DHALF]
        wb = jnp.zeros((DHALF, DPAD), jnp.float32).at[:d - DHALF].set(w16[DHALF:])
        return wa, wb

    wsa1, wsb1 = wsplit(W_self1)
    wna1, wnb1 = wsplit(W_neigh1)
    wsa2, wsb2 = wsplit(W_self2)
    wna2, wnb2 = wsplit(W_neigh2)
    bp1 = jnp.zeros((DPAD,), jnp.float32).at[:d].set(b1).at[d].set(1.0)
    bp2 = jnp.zeros((DPAD,), jnp.float32).at[:d].set(b2)
    badd1 = jnp.zeros((8, DPAD), jnp.float32).at[0].set(bp1)
    badd2 = jnp.zeros((8, DPAD), jnp.float32).at[0].set(bp2)
    zer = jnp.zeros((ZCH, DHALF), jnp.float32)

    agg = _make_agg(n_pad, n_edges)
    combine1 = _make_combine(n_pad, split_out=True)
    combine2 = _make_combine(n_pad, split_out=False)
    score = _make_score(n_pad, n_edges, d)

    psrc, pdst = pos_edge_index[0], pos_edge_index[1]
    nsrc, ndst = neg_edge_index[0], neg_edge_index[1]

    g1 = agg(xa, xb, psrc, pdst, zer)
    h1a, h1b = combine1(xa, xb, g1[0, 0], g1[1, 0], g1[0, 1], g1[1, 1],
                        wsa1, wsb1, wna1, wnb1, badd1)
    g2 = agg(h1a, h1b, psrc, pdst, zer)
    h2 = combine2(h1a, h1b, g2[0, 0], g2[1, 0], g2[0, 1], g2[1, 1],
                  wsa2, wsb2, wna2, wnb2, badd2)
    pos, neg = score(h2, psrc, pdst, nsrc, ndst)
    return pos.reshape(n_edges, 1), neg.reshape(n_edges, 1)


# speedup vs baseline: 40.0979x; 1.2863x over previous
import functools

import jax
import jax.numpy as jnp
from jax import lax
from jax.experimental import pallas as pl
from jax.experimental.pallas import tpu as pltpu
from jax.experimental.pallas import tpu_sc as plsc

NC = 2
NS = 16
NW = NC * NS
LANES = 16
DHALF = 8
DPAD = 16
ZCH = 272


def _mesh():
    return plsc.VectorSubcoreMesh(
        core_axis_name="c", subcore_axis_name="s", num_cores=NC, num_subcores=NS
    )


_SC_PARAMS = pltpu.CompilerParams(
    use_tc_tiling_on_sc=False, needs_layout_passes=False
)


def _make_agg(n_nodes, n_edges):
    epw = n_edges // NW
    ch = 4000
    nit = epw // ch
    rows_per_tile = n_nodes // NS
    nz = rows_per_tile // ZCH

    @functools.partial(
        pl.kernel,
        mesh=_mesh(),
        compiler_params=_SC_PARAMS,
        out_type=jax.ShapeDtypeStruct((NC, 2, n_nodes, DHALF), jnp.float32),
        scratch_types=[
            pltpu.VMEM((2, ch), jnp.int32),
            pltpu.VMEM((2, ch), jnp.int32),
            pltpu.VMEM((2, ch, DHALF), jnp.float32),
            pltpu.VMEM_SHARED((n_nodes, DHALF), jnp.float32),
            pltpu.SemaphoreType.DMA,
            pltpu.SemaphoreType.DMA,
        ],
    )
    def agg(xa_hbm, xb_hbm, src_hbm, dst_hbm, zer_hbm, out_hbm,
            sidx, didx, rows, acc, sem0, sem1):
        cid = lax.axis_index("c")
        sid = lax.axis_index("s")
        wid = sid * NC + cid
        row0 = sid * rows_per_tile
        base0 = wid * epw

        for ph, x_hbm in enumerate((xa_hbm, xb_hbm)):
            sems = (sem0, sem1)

            def issue(slot, i, sem):
                base = pl.multiple_of(base0 + i * ch, 8)
                pltpu.sync_copy(src_hbm.at[pl.ds(base, ch)], sidx.at[slot])
                pltpu.sync_copy(dst_hbm.at[pl.ds(base, ch)], didx.at[slot])
                pltpu.async_copy(x_hbm.at[sidx.at[slot]], rows.at[slot], sem)

            def drain(slot, sem):
                pltpu.make_async_copy(
                    x_hbm.at[sidx.at[slot]], rows.at[slot], sem).wait()
                pltpu.sync_copy(rows.at[slot], acc.at[didx.at[slot]], add=True)

            def zacc(i, _):
                pltpu.sync_copy(zer_hbm, acc.at[pl.ds(row0 + i * ZCH, ZCH)])
                return 0
            lax.fori_loop(0, nz, zacc, 0)
            plsc.subcore_barrier()

            issue(0, 0, sem0)

            def pair(j, _):
                issue(1, 2 * j + 1, sem1)
                drain(0, sem0)

                @pl.when(j < nit // 2 - 1)
                def _():
                    issue(0, 2 * j + 2, sem0)
                drain(1, sem1)
                return 0
            lax.fori_loop(0, nit // 2, pair, 0)
            plsc.subcore_barrier()

            pltpu.sync_copy(
                acc.at[pl.ds(row0, rows_per_tile)],
                out_hbm.at[cid, ph, pl.ds(row0, rows_per_tile)],
            )
            plsc.subcore_barrier()

    return agg


def _make_combine(n_nodes, split_out):
    blk = n_nodes // 32
    grid = n_nodes // blk
    bspec8 = pl.BlockSpec((blk, DHALF), lambda i: (i, 0))
    wspec = pl.BlockSpec((DHALF, DPAD), lambda i: (0, 0))

    def body(xa_ref, xb_ref, a0a_ref, a1a_ref, a0b_ref, a1b_ref,
             wsa_ref, wsb_ref, wna_ref, wnb_ref, b_ref, *out_refs):
        comb_a = a0a_ref[...] + a1a_ref[...]
        comb_b = a0b_ref[...] + a1b_ref[...]
        deg = jnp.maximum(comb_b[:, 2:3], 1.0)
        dot = functools.partial(jnp.dot, preferred_element_type=jnp.float32)
        h = (
            dot(xa_ref[...], wsa_ref[...])
            + dot(xb_ref[...], wsb_ref[...])
            + dot(comb_a / deg, wna_ref[...])
            + dot(comb_b / deg, wnb_ref[...])
            + b_ref[0:1, :]
        )
        if split_out:
            out_refs[0][...] = h[:, :DHALF]
            out_refs[1][...] = h[:, DHALF:]
        else:
            out_refs[0][...] = h

    if split_out:
        out_shape = (
            jax.ShapeDtypeStruct((n_nodes, DHALF), jnp.float32),
            jax.ShapeDtypeStruct((n_nodes, DHALF), jnp.float32),
        )
        out_specs = (bspec8, bspec8)
    else:
        out_shape = jax.ShapeDtypeStruct((n_nodes, DPAD), jnp.float32)
        out_specs = pl.BlockSpec((blk, DPAD), lambda i: (i, 0))

    return pl.pallas_call(
        body,
        grid=(grid,),
        in_specs=[bspec8] * 6 + [wspec] * 4
        + [pl.BlockSpec((8, DPAD), lambda i: (0, 0))],
        out_specs=out_specs,
        out_shape=out_shape,
    )


def _make_score(n_nodes, n_edges, n_feat):
    epw = n_edges // NW
    ch = 800
    nit = epw // ch
    ngrp = ch // LANES

    @functools.partial(
        pl.kernel,
        mesh=_mesh(),
        compiler_params=_SC_PARAMS,
        out_type=(
            jax.ShapeDtypeStruct((n_edges,), jnp.float32),
            jax.ShapeDtypeStruct((n_edges,), jnp.float32),
        ),
        scratch_types=[
            pltpu.VMEM((2, ch), jnp.int32),
            pltpu.VMEM((2, ch), jnp.int32),
            pltpu.VMEM((2, ch, DPAD), jnp.float32),
            pltpu.VMEM((2, ch, DPAD), jnp.float32),
            pltpu.VMEM((ch,), jnp.float32),
            pltpu.SemaphoreType.DMA,
            pltpu.SemaphoreType.DMA,
            pltpu.SemaphoreType.DMA,
            pltpu.SemaphoreType.DMA,
        ],
    )
    def score(h_hbm, pu_hbm, pv_hbm, nu_hbm, nv_hbm, pos_out, neg_out,
              uidx, vidx, urows, vrows, sc, semu0, semv0, semu1, semv1):
        cid = lax.axis_index("c")
        sid = lax.axis_index("s")
        wid = sid * NC + cid
        base0 = wid * epw
        lane = lax.iota(jnp.int32, 16)
        sems = ((semu0, semv0), (semu1, semv1))

        def run(u_hbm, v_hbm, out_hbm):
            def issue(slot, i):
                base = pl.multiple_of(base0 + i * ch, 8)
                pltpu.sync_copy(u_hbm.at[pl.ds(base, ch)], uidx.at[slot])
                pltpu.sync_copy(v_hbm.at[pl.ds(base, ch)], vidx.at[slot])
                pltpu.async_copy(h_hbm.at[uidx.at[slot]], urows.at[slot],
                                 sems[slot][0])
                pltpu.async_copy(h_hbm.at[vidx.at[slot]], vrows.at[slot],
                                 sems[slot][1])

            def drain(slot, i):
                pltpu.make_async_copy(
                    h_hbm.at[uidx.at[slot]], urows.at[slot], sems[slot][0]).wait()
                pltpu.make_async_copy(
                    h_hbm.at[vidx.at[slot]], vrows.at[slot], sems[slot][1]).wait()
                ur = urows.at[slot]
                vr = vrows.at[slot]

                def grp(g, _):
                    evec = lane + g * LANES
                    acc = jnp.zeros((16,), jnp.float32)
                    for l in range(n_feat):
                        lvec = jnp.full((16,), l, jnp.int32)
                        gu = plsc.load_gather(ur, [evec, lvec])
                        gv = plsc.load_gather(vr, [evec, lvec])
                        acc = acc + gu * gv
                    sc[pl.ds(pl.multiple_of(g * LANES, 8), LANES)] = acc
                    return 0
                lax.fori_loop(0, ngrp, grp, 0, unroll=2)
                base = pl.multiple_of(base0 + i * ch, 8)
                pltpu.sync_copy(sc, out_hbm.at[pl.ds(base, ch)])

            issue(0, 0)

            def pair(j, _):
                issue(1, 2 * j + 1)
                drain(0, 2 * j)

                @pl.when(j < nit // 2 - 1)
                def _():
                    issue(0, 2 * j + 2)
                drain(1, 2 * j + 1)
                return 0
            lax.fori_loop(0, nit // 2, pair, 0)

        run(pu_hbm, pv_hbm, pos_out)
        run(nu_hbm, nv_hbm, neg_out)

    return score


def kernel(node_features, pos_edge_index, neg_edge_index,
           W_self1, W_neigh1, b1, W_self2, W_neigh2, b2):
    n_nodes, d = node_features.shape
    n_edges = pos_edge_index.shape[1]
    n_pad = -(-n_nodes // 128) * 128

    xa = jnp.zeros((n_pad, DHALF), jnp.float32).at[:n_nodes].set(
        node_features[:, :DHALF])
    xb = jnp.zeros((n_pad, DHALF), jnp.float32)
    xb = xb.at[:n_nodes, :d - DHALF].set(node_features[:, DHALF:])
    xb = xb.at[:n_nodes, 2].set(1.0)

    def wsplit(w):
        w16 = jnp.zeros((d, DPAD), jnp.float32).at[:, :d].set(w)
        wa = w16[:DHALF]
        wb = jnp.zeros((DHALF, DPAD), jnp.float32).at[:d - DHALF].set(w16[DHALF:])
        return wa, wb

    wsa1, wsb1 = wsplit(W_self1)
    wna1, wnb1 = wsplit(W_neigh1)
    wsa2, wsb2 = wsplit(W_self2)
    wna2, wnb2 = wsplit(W_neigh2)
    bp1 = jnp.zeros((DPAD,), jnp.float32).at[:d].set(b1).at[d].set(1.0)
    bp2 = jnp.zeros((DPAD,), jnp.float32).at[:d].set(b2)
    badd1 = jnp.zeros((8, DPAD), jnp.float32).at[0].set(bp1)
    badd2 = jnp.zeros((8, DPAD), jnp.float32).at[0].set(bp2)
    zer = jnp.zeros((ZCH, DHALF), jnp.float32)

    agg = _make_agg(n_pad, n_edges)
    combine1 = _make_combine(n_pad, split_out=True)
    combine2 = _make_combine(n_pad, split_out=False)
    score = _make_score(n_pad, n_edges, d)

    psrc, pdst = pos_edge_index[0], pos_edge_index[1]
    nsrc, ndst = neg_edge_index[0], neg_edge_index[1]

    g1 = agg(xa, xb, psrc, pdst, zer)
    h1a, h1b = combine1(xa, xb, g1[0, 0], g1[1, 0], g1[0, 1], g1[1, 1],
                        wsa1, wsb1, wna1, wnb1, badd1)
    g2 = agg(h1a, h1b, psrc, pdst, zer)
    h2 = combine2(h1a, h1b, g2[0, 0], g2[1, 0], g2[0, 1], g2[1, 1],
                  wsa2, wsb2, wna2, wnb2, badd2)
    pos, neg = score(h2, psrc, pdst, nsrc, ndst)
    return pos.reshape(n_edges, 1), neg.reshape(n_edges, 1)


# speedup vs baseline: 40.8096x; 1.0177x over previous
import functools

import jax
import jax.numpy as jnp
from jax import lax
from jax.experimental import pallas as pl
from jax.experimental.pallas import tpu as pltpu
from jax.experimental.pallas import tpu_sc as plsc

NC = 2
NS = 16
NW = NC * NS
LANES = 16
DHALF = 8
DPAD = 16
ZCH = 272


def _mesh():
    return plsc.VectorSubcoreMesh(
        core_axis_name="c", subcore_axis_name="s", num_cores=NC, num_subcores=NS
    )


_SC_PARAMS = pltpu.CompilerParams(
    use_tc_tiling_on_sc=False, needs_layout_passes=False
)


def _make_agg(n_nodes, n_edges):
    epw = n_edges // NW
    ch = 4000
    nit = epw // ch
    rows_per_tile = n_nodes // NS
    nz = rows_per_tile // ZCH

    @functools.partial(
        pl.kernel,
        mesh=_mesh(),
        compiler_params=_SC_PARAMS,
        out_type=jax.ShapeDtypeStruct((NC, 2, n_nodes, DHALF), jnp.float32),
        scratch_types=[
            pltpu.VMEM((2, ch), jnp.int32),
            pltpu.VMEM((2, ch), jnp.int32),
            pltpu.VMEM((2, ch, DHALF), jnp.float32),
            pltpu.VMEM_SHARED((n_nodes, DHALF), jnp.float32),
            pltpu.SemaphoreType.DMA,
            pltpu.SemaphoreType.DMA,
        ],
    )
    def agg(xa_hbm, xb_hbm, src_hbm, dst_hbm, zer_hbm, out_hbm,
            sidx, didx, rows, acc, sem0, sem1):
        cid = lax.axis_index("c")
        sid = lax.axis_index("s")
        wid = sid * NC + cid
        row0 = sid * rows_per_tile
        base0 = wid * epw

        for ph, x_hbm in enumerate((xa_hbm, xb_hbm)):
            sems = (sem0, sem1)

            def issue(slot, i, sem):
                base = pl.multiple_of(base0 + i * ch, 8)
                pltpu.sync_copy(src_hbm.at[pl.ds(base, ch)], sidx.at[slot])
                pltpu.sync_copy(dst_hbm.at[pl.ds(base, ch)], didx.at[slot])
                pltpu.async_copy(x_hbm.at[sidx.at[slot]], rows.at[slot], sem)

            def drain(slot, sem):
                pltpu.make_async_copy(
                    x_hbm.at[sidx.at[slot]], rows.at[slot], sem).wait()
                pltpu.sync_copy(rows.at[slot], acc.at[didx.at[slot]], add=True)

            def zacc(i, _):
                pltpu.sync_copy(zer_hbm, acc.at[pl.ds(row0 + i * ZCH, ZCH)])
                return 0
            lax.fori_loop(0, nz, zacc, 0)
            plsc.subcore_barrier()

            issue(0, 0, sem0)

            def pair(j, _):
                issue(1, 2 * j + 1, sem1)
                drain(0, sem0)

                @pl.when(j < nit // 2 - 1)
                def _():
                    issue(0, 2 * j + 2, sem0)
                drain(1, sem1)
                return 0
            lax.fori_loop(0, nit // 2, pair, 0)
            plsc.subcore_barrier()

            pltpu.sync_copy(
                acc.at[pl.ds(row0, rows_per_tile)],
                out_hbm.at[cid, ph, pl.ds(row0, rows_per_tile)],
            )
            plsc.subcore_barrier()

    return agg


def _make_combine(n_nodes, split_out):
    blk = n_nodes // 32
    grid = n_nodes // blk
    bspec8 = pl.BlockSpec((blk, DHALF), lambda i: (i, 0))
    wspec = pl.BlockSpec((DHALF, DPAD), lambda i: (0, 0))

    def body(xa_ref, xb_ref, a0a_ref, a1a_ref, a0b_ref, a1b_ref,
             wsa_ref, wsb_ref, wna_ref, wnb_ref, b_ref, *out_refs):
        comb_a = a0a_ref[...] + a1a_ref[...]
        comb_b = a0b_ref[...] + a1b_ref[...]
        deg = jnp.maximum(comb_b[:, 2:3], 1.0)
        dot = functools.partial(jnp.dot, preferred_element_type=jnp.float32)
        h = (
            dot(xa_ref[...], wsa_ref[...])
            + dot(xb_ref[...], wsb_ref[...])
            + dot(comb_a / deg, wna_ref[...])
            + dot(comb_b / deg, wnb_ref[...])
            + b_ref[0:1, :]
        )
        if split_out:
            out_refs[0][...] = h[:, :DHALF]
            out_refs[1][...] = h[:, DHALF:]
        else:
            out_refs[0][...] = h

    if split_out:
        out_shape = (
            jax.ShapeDtypeStruct((n_nodes, DHALF), jnp.float32),
            jax.ShapeDtypeStruct((n_nodes, DHALF), jnp.float32),
        )
        out_specs = (bspec8, bspec8)
    else:
        out_shape = jax.ShapeDtypeStruct((n_nodes, DPAD), jnp.float32)
        out_specs = pl.BlockSpec((blk, DPAD), lambda i: (i, 0))

    return pl.pallas_call(
        body,
        grid=(grid,),
        in_specs=[bspec8] * 6 + [wspec] * 4
        + [pl.BlockSpec((8, DPAD), lambda i: (0, 0))],
        out_specs=out_specs,
        out_shape=out_shape,
    )


def _make_score(n_nodes, n_edges, n_feat):
    epw = n_edges // NW
    ch = 800
    nit = epw // ch
    ngrp = ch // LANES

    @functools.partial(
        pl.kernel,
        mesh=_mesh(),
        compiler_params=_SC_PARAMS,
        out_type=(
            jax.ShapeDtypeStruct((n_edges,), jnp.float32),
            jax.ShapeDtypeStruct((n_edges,), jnp.float32),
        ),
        scratch_types=[
            pltpu.VMEM((2, ch), jnp.int32),
            pltpu.VMEM((2, ch), jnp.int32),
            pltpu.VMEM((2, ch, DPAD), jnp.float32),
            pltpu.VMEM((2, ch, DPAD), jnp.float32),
            pltpu.VMEM((ch,), jnp.float32),
            pltpu.SemaphoreType.DMA,
            pltpu.SemaphoreType.DMA,
            pltpu.SemaphoreType.DMA,
            pltpu.SemaphoreType.DMA,
        ],
    )
    def score(h_hbm, pu_hbm, pv_hbm, nu_hbm, nv_hbm, pos_out, neg_out,
              uidx, vidx, urows, vrows, sc, semu0, semv0, semu1, semv1):
        cid = lax.axis_index("c")
        sid = lax.axis_index("s")
        wid = sid * NC + cid
        base0 = wid * epw
        lane = lax.iota(jnp.int32, 16)
        sems = ((semu0, semv0), (semu1, semv1))

        def run(u_hbm, v_hbm, out_hbm):
            def issue(slot, i):
                base = pl.multiple_of(base0 + i * ch, 8)
                pltpu.sync_copy(u_hbm.at[pl.ds(base, ch)], uidx.at[slot])
                pltpu.sync_copy(v_hbm.at[pl.ds(base, ch)], vidx.at[slot])
                pltpu.async_copy(h_hbm.at[uidx.at[slot]], urows.at[slot],
                                 sems[slot][0])
                pltpu.async_copy(h_hbm.at[vidx.at[slot]], vrows.at[slot],
                                 sems[slot][1])

            def drain(slot, i):
                pltpu.make_async_copy(
                    h_hbm.at[uidx.at[slot]], urows.at[slot], sems[slot][0]).wait()
                pltpu.make_async_copy(
                    h_hbm.at[vidx.at[slot]], vrows.at[slot], sems[slot][1]).wait()
                ur = urows.at[slot]
                vr = vrows.at[slot]

                def grp(g, _):
                    evec = lane + g * LANES
                    prods = []
                    for l in range(n_feat):
                        lvec = jnp.full((16,), l, jnp.int32)
                        gu = plsc.load_gather(ur, [evec, lvec])
                        gv = plsc.load_gather(vr, [evec, lvec])
                        prods.append(gu * gv)
                    while len(prods) > 1:
                        prods = [a + b for a, b in zip(prods[::2], prods[1::2])] \
                            + ([prods[-1]] if len(prods) % 2 else [])
                    sc[pl.ds(pl.multiple_of(g * LANES, 8), LANES)] = prods[0]
                    return 0
                lax.fori_loop(0, ngrp, grp, 0, unroll=2)
                base = pl.multiple_of(base0 + i * ch, 8)
                pltpu.sync_copy(sc, out_hbm.at[pl.ds(base, ch)])

            issue(0, 0)

            def pair(j, _):
                issue(1, 2 * j + 1)
                drain(0, 2 * j)

                @pl.when(j < nit // 2 - 1)
                def _():
                    issue(0, 2 * j + 2)
                drain(1, 2 * j + 1)
                return 0
            lax.fori_loop(0, nit // 2, pair, 0)

        run(pu_hbm, pv_hbm, pos_out)
        run(nu_hbm, nv_hbm, neg_out)

    return score


def kernel(node_features, pos_edge_index, neg_edge_index,
           W_self1, W_neigh1, b1, W_self2, W_neigh2, b2):
    n_nodes, d = node_features.shape
    n_edges = pos_edge_index.shape[1]
    n_pad = -(-n_nodes // 128) * 128

    xa = jnp.zeros((n_pad, DHALF), jnp.float32).at[:n_nodes].set(
        node_features[:, :DHALF])
    xb = jnp.zeros((n_pad, DHALF), jnp.float32)
    xb = xb.at[:n_nodes, :d - DHALF].set(node_features[:, DHALF:])
    xb = xb.at[:n_nodes, 2].set(1.0)

    def wsplit(w):
        w16 = jnp.zeros((d, DPAD), jnp.float32).at[:, :d].set(w)
        wa = w16[:DHALF]
        wb = jnp.zeros((DHALF, DPAD), jnp.float32).at[:d - DHALF].set(w16[DHALF:])
        return wa, wb

    wsa1, wsb1 = wsplit(W_self1)
    wna1, wnb1 = wsplit(W_neigh1)
    wsa2, wsb2 = wsplit(W_self2)
    wna2, wnb2 = wsplit(W_neigh2)
    bp1 = jnp.zeros((DPAD,), jnp.float32).at[:d].set(b1).at[d].set(1.0)
    bp2 = jnp.zeros((DPAD,), jnp.float32).at[:d].set(b2)
    badd1 = jnp.zeros((8, DPAD), jnp.float32).at[0].set(bp1)
    badd2 = jnp.zeros((8, DPAD), jnp.float32).at[0].set(bp2)
    zer = jnp.zeros((ZCH, DHALF), jnp.float32)

    agg = _make_agg(n_pad, n_edges)
    combine1 = _make_combine(n_pad, split_out=True)
    combine2 = _make_combine(n_pad, split_out=False)
    score = _make_score(n_pad, n_edges, d)

    psrc, pdst = pos_edge_index[0], pos_edge_index[1]
    nsrc, ndst = neg_edge_index[0], neg_edge_index[1]

    g1 = agg(xa, xb, psrc, pdst, zer)
    h1a, h1b = combine1(xa, xb, g1[0, 0], g1[1, 0], g1[0, 1], g1[1, 1],
                        wsa1, wsb1, wna1, wnb1, badd1)
    g2 = agg(h1a, h1b, psrc, pdst, zer)
    h2 = combine2(h1a, h1b, g2[0, 0], g2[1, 0], g2[0, 1], g2[1, 1],
                  wsa2, wsb2, wna2, wnb2, badd2)
    pos, neg = score(h2, psrc, pdst, nsrc, ndst)
    return pos.reshape(n_edges, 1), neg.reshape(n_edges, 1)


# speedup vs baseline: 46.3259x; 1.1352x over previous
import functools

import jax
import jax.numpy as jnp
from jax import lax
from jax.experimental import pallas as pl
from jax.experimental.pallas import tpu as pltpu
from jax.experimental.pallas import tpu_sc as plsc

NC = 2
NS = 16
NW = NC * NS
LANES = 16
DHALF = 8
DPAD = 16
ZCH = 272


def _mesh():
    return plsc.VectorSubcoreMesh(
        core_axis_name="c", subcore_axis_name="s", num_cores=NC, num_subcores=NS
    )


_SC_PARAMS = pltpu.CompilerParams(
    use_tc_tiling_on_sc=False, needs_layout_passes=False
)


def _make_agg(n_nodes, n_edges):
    epw = n_edges // NW
    ch = 4000
    nit = epw // ch
    rows_per_tile = n_nodes // NS
    nz = rows_per_tile // ZCH

    @functools.partial(
        pl.kernel,
        mesh=_mesh(),
        compiler_params=_SC_PARAMS,
        out_type=jax.ShapeDtypeStruct((NC, 2, n_nodes, DHALF), jnp.float32),
        scratch_types=[
            pltpu.VMEM((2, ch), jnp.int32),
            pltpu.VMEM((2, ch), jnp.int32),
            pltpu.VMEM((2, ch, DHALF), jnp.float32),
            pltpu.VMEM_SHARED((n_nodes, DHALF), jnp.float32),
            [pltpu.SemaphoreType.DMA] * 2,
            [pltpu.SemaphoreType.DMA] * 2,
        ],
    )
    def agg(xa_hbm, xb_hbm, src_hbm, dst_hbm, zer_hbm, out_hbm,
            sidx, didx, rows, acc, sem_i, sem_g):
        cid = lax.axis_index("c")
        sid = lax.axis_index("s")
        wid = sid * NC + cid
        row0 = sid * rows_per_tile
        base0 = wid * epw

        for ph, x_hbm in enumerate((xa_hbm, xb_hbm)):

            def idx_issue(isl, i):
                base = pl.multiple_of(base0 + i * ch, 8)
                pltpu.async_copy(src_hbm.at[pl.ds(base, ch)], sidx.at[isl],
                                 sem_i[isl])
                pltpu.async_copy(dst_hbm.at[pl.ds(base, ch)], didx.at[isl],
                                 sem_i[isl])

            def idx_wait(isl):
                pltpu.make_async_copy(
                    src_hbm.at[pl.ds(base0, ch)], sidx.at[isl], sem_i[isl]).wait()
                pltpu.make_async_copy(
                    dst_hbm.at[pl.ds(base0, ch)], didx.at[isl], sem_i[isl]).wait()

            def gath_issue(slot):
                pltpu.async_copy(x_hbm.at[sidx.at[slot]], rows.at[slot],
                                 sem_g[slot])

            def gath_wait(slot):
                pltpu.make_async_copy(
                    x_hbm.at[sidx.at[slot]], rows.at[slot], sem_g[slot]).wait()

            def scat(slot):
                pltpu.sync_copy(rows.at[slot], acc.at[didx.at[slot]], add=True)

            def zacc(i, _):
                pltpu.sync_copy(zer_hbm, acc.at[pl.ds(row0 + i * ZCH, ZCH)])
                return 0
            lax.fori_loop(0, nz, zacc, 0)
            plsc.subcore_barrier()

            idx_issue(0, 0)
            idx_issue(1, 1)
            idx_wait(0)
            gath_issue(0)

            def pair(j, _):
                c0 = 2 * j
                c1 = c0 + 1
                idx_wait(1)
                gath_issue(1)
                gath_wait(0)
                scat(0)

                @pl.when(c0 + 2 < nit)
                def _():
                    idx_issue(0, c0 + 2)
                gath_wait(1)
                scat(1)

                @pl.when(c1 + 2 < nit)
                def _():
                    idx_issue(1, c1 + 2)

                @pl.when(c0 + 2 < nit)
                def _():
                    idx_wait(0)
                    gath_issue(0)
                return 0
            lax.fori_loop(0, nit // 2, pair, 0)
            plsc.subcore_barrier()

            pltpu.sync_copy(
                acc.at[pl.ds(row0, rows_per_tile)],
                out_hbm.at[cid, ph, pl.ds(row0, rows_per_tile)],
            )
            plsc.subcore_barrier()

    return agg


def _make_combine(n_nodes, split_out):
    blk = n_nodes // 32
    grid = n_nodes // blk
    bspec8 = pl.BlockSpec((blk, DHALF), lambda i: (i, 0))
    wspec = pl.BlockSpec((DHALF, DPAD), lambda i: (0, 0))

    def body(xa_ref, xb_ref, a0a_ref, a1a_ref, a0b_ref, a1b_ref,
             wsa_ref, wsb_ref, wna_ref, wnb_ref, b_ref, *out_refs):
        comb_a = a0a_ref[...] + a1a_ref[...]
        comb_b = a0b_ref[...] + a1b_ref[...]
        deg = jnp.maximum(comb_b[:, 2:3], 1.0)
        dot = functools.partial(jnp.dot, preferred_element_type=jnp.float32)
        h = (
            dot(xa_ref[...], wsa_ref[...])
            + dot(xb_ref[...], wsb_ref[...])
            + dot(comb_a / deg, wna_ref[...])
            + dot(comb_b / deg, wnb_ref[...])
            + b_ref[0:1, :]
        )
        if split_out:
            out_refs[0][...] = h[:, :DHALF]
            out_refs[1][...] = h[:, DHALF:]
        else:
            out_refs[0][...] = h

    if split_out:
        out_shape = (
            jax.ShapeDtypeStruct((n_nodes, DHALF), jnp.float32),
            jax.ShapeDtypeStruct((n_nodes, DHALF), jnp.float32),
        )
        out_specs = (bspec8, bspec8)
    else:
        out_shape = jax.ShapeDtypeStruct((n_nodes, DPAD), jnp.float32)
        out_specs = pl.BlockSpec((blk, DPAD), lambda i: (i, 0))

    return pl.pallas_call(
        body,
        grid=(grid,),
        in_specs=[bspec8] * 6 + [wspec] * 4
        + [pl.BlockSpec((8, DPAD), lambda i: (0, 0))],
        out_specs=out_specs,
        out_shape=out_shape,
    )


def _make_score(n_nodes, n_edges, n_feat):
    epw = n_edges // NW
    ch = 800
    nit = epw // ch
    ngrp = ch // LANES

    @functools.partial(
        pl.kernel,
        mesh=_mesh(),
        compiler_params=_SC_PARAMS,
        out_type=(
            jax.ShapeDtypeStruct((n_edges,), jnp.float32),
            jax.ShapeDtypeStruct((n_edges,), jnp.float32),
        ),
        scratch_types=[
            pltpu.VMEM((2, ch), jnp.int32),
            pltpu.VMEM((2, ch), jnp.int32),
            pltpu.VMEM((2, ch, DPAD), jnp.float32),
            pltpu.VMEM((2, ch, DPAD), jnp.float32),
            pltpu.VMEM((2, ch), jnp.float32),
            [pltpu.SemaphoreType.DMA] * 2,
            [pltpu.SemaphoreType.DMA] * 2,
            [pltpu.SemaphoreType.DMA] * 2,
        ],
    )
    def score(h_hbm, pu_hbm, pv_hbm, nu_hbm, nv_hbm, pos_out, neg_out,
              uidx, vidx, urows, vrows, sc, sem_i, sem_g, sem_o):
        cid = lax.axis_index("c")
        sid = lax.axis_index("s")
        wid = sid * NC + cid
        base0 = wid * epw
        lane = lax.iota(jnp.int32, 16)

        def run(u_hbm, v_hbm, out_hbm):
            def idx_issue(isl, i):
                base = pl.multiple_of(base0 + i * ch, 8)
                pltpu.async_copy(u_hbm.at[pl.ds(base, ch)], uidx.at[isl],
                                 sem_i[isl])
                pltpu.async_copy(v_hbm.at[pl.ds(base, ch)], vidx.at[isl],
                                 sem_i[isl])

            def idx_wait(isl):
                pltpu.make_async_copy(
                    u_hbm.at[pl.ds(base0, ch)], uidx.at[isl], sem_i[isl]).wait()
                pltpu.make_async_copy(
                    v_hbm.at[pl.ds(base0, ch)], vidx.at[isl], sem_i[isl]).wait()

            def gath_issue(slot, isl):
                pltpu.async_copy(h_hbm.at[uidx.at[isl]], urows.at[slot],
                                 sem_g[slot])
                pltpu.async_copy(h_hbm.at[vidx.at[isl]], vrows.at[slot],
                                 sem_g[slot])

            def gath_wait(slot, isl):
                pltpu.make_async_copy(
                    h_hbm.at[uidx.at[isl]], urows.at[slot], sem_g[slot]).wait()
                pltpu.make_async_copy(
                    h_hbm.at[vidx.at[isl]], vrows.at[slot], sem_g[slot]).wait()

            def out_wait(slot):
                pltpu.make_async_copy(
                    sc.at[slot], out_hbm.at[pl.ds(base0, ch)], sem_o[slot]).wait()

            def compute(slot, i):
                ur = urows.at[slot]
                vr = vrows.at[slot]
                scs = sc.at[slot]

                def grp(g, _):
                    evec = lane + g * LANES
                    prods = []
                    for l in range(n_feat):
                        lvec = jnp.full((16,), l, jnp.int32)
                        gu = plsc.load_gather(ur, [evec, lvec])
                        gv = plsc.load_gather(vr, [evec, lvec])
                        prods.append(gu * gv)
                    while len(prods) > 1:
                        prods = [a + b for a, b in zip(prods[::2], prods[1::2])] \
                            + ([prods[-1]] if len(prods) % 2 else [])
                    scs[pl.ds(pl.multiple_of(g * LANES, 8), LANES)] = prods[0]
                    return 0
                lax.fori_loop(0, ngrp, grp, 0, unroll=2)
                base = pl.multiple_of(base0 + i * ch, 8)
                pltpu.async_copy(scs, out_hbm.at[pl.ds(base, ch)], sem_o[slot])

            idx_issue(0, 0)
            idx_issue(1, 1)
            idx_wait(0)
            gath_issue(0, 0)

            def pair(j, _):
                c0 = 2 * j
                c1 = c0 + 1
                idx_wait(1)
                gath_issue(1, 1)
                gath_wait(0, 0)

                @pl.when(c0 + 2 < nit)
                def _():
                    idx_issue(0, c0 + 2)

                @pl.when(j > 0)
                def _():
                    out_wait(0)
                compute(0, c0)

                @pl.when(c0 + 2 < nit)
                def _():
                    idx_wait(0)
                    gath_issue(0, 0)
                gath_wait(1, 1)

                @pl.when(c1 + 2 < nit)
                def _():
                    idx_issue(1, c1 + 2)

                @pl.when(j > 0)
                def _():
                    out_wait(1)
                compute(1, c1)
                return 0
            lax.fori_loop(0, nit // 2, pair, 0)
            out_wait(0)
            out_wait(1)

        run(pu_hbm, pv_hbm, pos_out)
        run(nu_hbm, nv_hbm, neg_out)

    return score


def kernel(node_features, pos_edge_index, neg_edge_index,
           W_self1, W_neigh1, b1, W_self2, W_neigh2, b2):
    n_nodes, d = node_features.shape
    n_edges = pos_edge_index.shape[1]
    n_pad = -(-n_nodes // 128) * 128

    xa = jnp.zeros((n_pad, DHALF), jnp.float32).at[:n_nodes].set(
        node_features[:, :DHALF])
    xb = jnp.zeros((n_pad, DHALF), jnp.float32)
    xb = xb.at[:n_nodes, :d - DHALF].set(node_features[:, DHALF:])
    xb = xb.at[:n_nodes, 2].set(1.0)

    def wsplit(w):
        w16 = jnp.zeros((d, DPAD), jnp.float32).at[:, :d].set(w)
        wa = w16[:DHALF]
        wb = jnp.zeros((DHALF, DPAD), jnp.float32).at[:d - DHALF].set(w16[DHALF:])
        return wa, wb

    wsa1, wsb1 = wsplit(W_self1)
    wna1, wnb1 = wsplit(W_neigh1)
    wsa2, wsb2 = wsplit(W_self2)
    wna2, wnb2 = wsplit(W_neigh2)
    bp1 = jnp.zeros((DPAD,), jnp.float32).at[:d].set(b1).at[d].set(1.0)
    bp2 = jnp.zeros((DPAD,), jnp.float32).at[:d].set(b2)
    badd1 = jnp.zeros((8, DPAD), jnp.float32).at[0].set(bp1)
    badd2 = jnp.zeros((8, DPAD), jnp.float32).at[0].set(bp2)
    zer = jnp.zeros((ZCH, DHALF), jnp.float32)

    agg = _make_agg(n_pad, n_edges)
    combine1 = _make_combine(n_pad, split_out=True)
    combine2 = _make_combine(n_pad, split_out=False)
    score = _make_score(n_pad, n_edges, d)

    psrc, pdst = pos_edge_index[0], pos_edge_index[1]
    nsrc, ndst = neg_edge_index[0], neg_edge_index[1]

    g1 = agg(xa, xb, psrc, pdst, zer)
    h1a, h1b = combine1(xa, xb, g1[0, 0], g1[1, 0], g1[0, 1], g1[1, 1],
                        wsa1, wsb1, wna1, wnb1, badd1)
    g2 = agg(h1a, h1b, psrc, pdst, zer)
    h2 = combine2(h1a, h1b, g2[0, 0], g2[1, 0], g2[0, 1], g2[1, 1],
                  wsa2, wsb2, wna2, wnb2, badd2)
    pos, neg = score(h2, psrc, pdst, nsrc, ndst)
    return pos.reshape(n_edges, 1), neg.reshape(n_edges, 1)


# speedup vs baseline: 49.4347x; 1.0671x over previous
import functools

import jax
import jax.numpy as jnp
from jax import lax
from jax.experimental import pallas as pl
from jax.experimental.pallas import tpu as pltpu
from jax.experimental.pallas import tpu_sc as plsc

NC = 2
NS = 16
NW = NC * NS
LANES = 16
DHALF = 8
DPAD = 16
ZCH = 272


def _mesh():
    return plsc.VectorSubcoreMesh(
        core_axis_name="c", subcore_axis_name="s", num_cores=NC, num_subcores=NS
    )


_SC_PARAMS = pltpu.CompilerParams(
    use_tc_tiling_on_sc=False, needs_layout_passes=False
)


def _make_agg(n_nodes, n_edges):
    epw = n_edges // NW
    ch = 4000
    nit = epw // ch
    rows_per_tile = n_nodes // NS
    nz = rows_per_tile // ZCH

    @functools.partial(
        pl.kernel,
        mesh=_mesh(),
        compiler_params=_SC_PARAMS,
        out_type=jax.ShapeDtypeStruct((NC, 2, n_nodes, DHALF), jnp.float32),
        scratch_types=[
            pltpu.VMEM((2, ch), jnp.int32),
            pltpu.VMEM((2, ch), jnp.int32),
            pltpu.VMEM((2, ch, DHALF), jnp.float32),
            pltpu.VMEM_SHARED((n_nodes, DHALF), jnp.float32),
            [pltpu.SemaphoreType.DMA] * 2,
            [pltpu.SemaphoreType.DMA] * 2,
        ],
    )
    def agg(xa_hbm, xb_hbm, src_hbm, dst_hbm, zer_hbm, out_hbm,
            sidx, didx, rows, acc, sem_i, sem_g):
        cid = lax.axis_index("c")
        sid = lax.axis_index("s")
        wid = sid * NC + cid
        row0 = sid * rows_per_tile
        base0 = wid * epw

        for ph, x_hbm in enumerate((xa_hbm, xb_hbm)):

            def idx_issue(isl, i):
                base = pl.multiple_of(base0 + i * ch, 8)
                pltpu.async_copy(src_hbm.at[pl.ds(base, ch)], sidx.at[isl],
                                 sem_i[isl])
                pltpu.async_copy(dst_hbm.at[pl.ds(base, ch)], didx.at[isl],
                                 sem_i[isl])

            def idx_wait(isl):
                pltpu.make_async_copy(
                    src_hbm.at[pl.ds(base0, ch)], sidx.at[isl], sem_i[isl]).wait()
                pltpu.make_async_copy(
                    dst_hbm.at[pl.ds(base0, ch)], didx.at[isl], sem_i[isl]).wait()

            def gath_issue(slot):
                pltpu.async_copy(x_hbm.at[sidx.at[slot]], rows.at[slot],
                                 sem_g[slot])

            def gath_wait(slot):
                pltpu.make_async_copy(
                    x_hbm.at[sidx.at[slot]], rows.at[slot], sem_g[slot]).wait()

            def scat(slot):
                pltpu.sync_copy(rows.at[slot], acc.at[didx.at[slot]], add=True)

            def zacc(i, _):
                pltpu.sync_copy(zer_hbm, acc.at[pl.ds(row0 + i * ZCH, ZCH)])
                return 0
            lax.fori_loop(0, nz, zacc, 0)
            plsc.subcore_barrier()

            idx_issue(0, 0)
            idx_issue(1, 1)
            idx_wait(0)
            gath_issue(0)

            def pair(j, _):
                c0 = 2 * j
                c1 = c0 + 1
                idx_wait(1)
                gath_issue(1)
                gath_wait(0)
                scat(0)

                @pl.when(c0 + 2 < nit)
                def _():
                    idx_issue(0, c0 + 2)
                gath_wait(1)
                scat(1)

                @pl.when(c1 + 2 < nit)
                def _():
                    idx_issue(1, c1 + 2)

                @pl.when(c0 + 2 < nit)
                def _():
                    idx_wait(0)
                    gath_issue(0)
                return 0
            lax.fori_loop(0, nit // 2, pair, 0)
            plsc.subcore_barrier()

            pltpu.sync_copy(
                acc.at[pl.ds(row0, rows_per_tile)],
                out_hbm.at[cid, ph, pl.ds(row0, rows_per_tile)],
            )
            plsc.subcore_barrier()

    return agg


def _make_combine(n_nodes, split_out):
    blk = n_nodes // 32
    grid = n_nodes // blk
    bspec8 = pl.BlockSpec((blk, DHALF), lambda i: (i, 0))
    wspec = pl.BlockSpec((DHALF, DPAD), lambda i: (0, 0))

    def body(xa_ref, xb_ref, g_ref,
             wsa_ref, wsb_ref, wna_ref, wnb_ref, b_ref, *out_refs):
        comb_a = g_ref[0, 0] + g_ref[1, 0]
        comb_b = g_ref[0, 1] + g_ref[1, 1]
        deg = jnp.maximum(comb_b[:, 2:3], 1.0)
        dot = functools.partial(jnp.dot, preferred_element_type=jnp.float32)
        h = (
            dot(xa_ref[...], wsa_ref[...])
            + dot(xb_ref[...], wsb_ref[...])
            + dot(comb_a / deg, wna_ref[...])
            + dot(comb_b / deg, wnb_ref[...])
            + b_ref[0:1, :]
        )
        if split_out:
            out_refs[0][...] = h[:, :DHALF]
            out_refs[1][...] = h[:, DHALF:]
        else:
            out_refs[0][...] = h

    if split_out:
        out_shape = (
            jax.ShapeDtypeStruct((n_nodes, DHALF), jnp.float32),
            jax.ShapeDtypeStruct((n_nodes, DHALF), jnp.float32),
        )
        out_specs = (bspec8, bspec8)
    else:
        out_shape = jax.ShapeDtypeStruct((n_nodes, DPAD), jnp.float32)
        out_specs = pl.BlockSpec((blk, DPAD), lambda i: (i, 0))

    return pl.pallas_call(
        body,
        grid=(grid,),
        in_specs=[bspec8] * 2
        + [pl.BlockSpec((NC, 2, blk, DHALF), lambda i: (0, 0, i, 0))]
        + [wspec] * 4
        + [pl.BlockSpec((8, DPAD), lambda i: (0, 0))],
        out_specs=out_specs,
        out_shape=out_shape,
    )


def _make_score(n_nodes, n_edges, n_feat):
    epw = n_edges // NW
    ch = 800
    nit = epw // ch
    ngrp = ch // LANES

    @functools.partial(
        pl.kernel,
        mesh=_mesh(),
        compiler_params=_SC_PARAMS,
        out_type=(
            jax.ShapeDtypeStruct((n_edges,), jnp.float32),
            jax.ShapeDtypeStruct((n_edges,), jnp.float32),
        ),
        scratch_types=[
            pltpu.VMEM((2, ch), jnp.int32),
            pltpu.VMEM((2, ch), jnp.int32),
            pltpu.VMEM((2, ch, DPAD), jnp.float32),
            pltpu.VMEM((2, ch, DPAD), jnp.float32),
            pltpu.VMEM((2, ch), jnp.float32),
            [pltpu.SemaphoreType.DMA] * 2,
            [pltpu.SemaphoreType.DMA] * 2,
            [pltpu.SemaphoreType.DMA] * 2,
        ],
    )
    def score(h_hbm, pu_hbm, pv_hbm, nu_hbm, nv_hbm, pos_out, neg_out,
              uidx, vidx, urows, vrows, sc, sem_i, sem_g, sem_o):
        cid = lax.axis_index("c")
        sid = lax.axis_index("s")
        wid = sid * NC + cid
        base0 = wid * epw
        lane = lax.iota(jnp.int32, 16)

        def run(u_hbm, v_hbm, out_hbm):
            def idx_issue(isl, i):
                base = pl.multiple_of(base0 + i * ch, 8)
                pltpu.async_copy(u_hbm.at[pl.ds(base, ch)], uidx.at[isl],
                                 sem_i[isl])
                pltpu.async_copy(v_hbm.at[pl.ds(base, ch)], vidx.at[isl],
                                 sem_i[isl])

            def idx_wait(isl):
                pltpu.make_async_copy(
                    u_hbm.at[pl.ds(base0, ch)], uidx.at[isl], sem_i[isl]).wait()
                pltpu.make_async_copy(
                    v_hbm.at[pl.ds(base0, ch)], vidx.at[isl], sem_i[isl]).wait()

            def gath_issue(slot, isl):
                pltpu.async_copy(h_hbm.at[uidx.at[isl]], urows.at[slot],
                                 sem_g[slot])
                pltpu.async_copy(h_hbm.at[vidx.at[isl]], vrows.at[slot],
                                 sem_g[slot])

            def gath_wait(slot, isl):
                pltpu.make_async_copy(
                    h_hbm.at[uidx.at[isl]], urows.at[slot], sem_g[slot]).wait()
                pltpu.make_async_copy(
                    h_hbm.at[vidx.at[isl]], vrows.at[slot], sem_g[slot]).wait()

            def out_wait(slot):
                pltpu.make_async_copy(
                    sc.at[slot], out_hbm.at[pl.ds(base0, ch)], sem_o[slot]).wait()

            def compute(slot, i):
                ur = urows.at[slot]
                vr = vrows.at[slot]
                scs = sc.at[slot]

                def grp(g, _):
                    evec = lane + g * LANES
                    prods = []
                    for l in range(n_feat):
                        lvec = jnp.full((16,), l, jnp.int32)
                        gu = plsc.load_gather(ur, [evec, lvec])
                        gv = plsc.load_gather(vr, [evec, lvec])
                        prods.append(gu * gv)
                    while len(prods) > 1:
                        prods = [a + b for a, b in zip(prods[::2], prods[1::2])] \
                            + ([prods[-1]] if len(prods) % 2 else [])
                    scs[pl.ds(pl.multiple_of(g * LANES, 8), LANES)] = prods[0]
                    return 0
                lax.fori_loop(0, ngrp, grp, 0, unroll=2)
                base = pl.multiple_of(base0 + i * ch, 8)
                pltpu.async_copy(scs, out_hbm.at[pl.ds(base, ch)], sem_o[slot])

            idx_issue(0, 0)
            idx_issue(1, 1)
            idx_wait(0)
            gath_issue(0, 0)

            def pair(j, _):
                c0 = 2 * j
                c1 = c0 + 1
                idx_wait(1)
                gath_issue(1, 1)
                gath_wait(0, 0)

                @pl.when(c0 + 2 < nit)
                def _():
                    idx_issue(0, c0 + 2)

                @pl.when(j > 0)
                def _():
                    out_wait(0)
                compute(0, c0)

                @pl.when(c0 + 2 < nit)
                def _():
                    idx_wait(0)
                    gath_issue(0, 0)
                gath_wait(1, 1)

                @pl.when(c1 + 2 < nit)
                def _():
                    idx_issue(1, c1 + 2)

                @pl.when(j > 0)
                def _():
                    out_wait(1)
                compute(1, c1)
                return 0
            lax.fori_loop(0, nit // 2, pair, 0)
            out_wait(0)
            out_wait(1)

        run(pu_hbm, pv_hbm, pos_out)
        run(nu_hbm, nv_hbm, neg_out)

    return score


def kernel(node_features, pos_edge_index, neg_edge_index,
           W_self1, W_neigh1, b1, W_self2, W_neigh2, b2):
    n_nodes, d = node_features.shape
    n_edges = pos_edge_index.shape[1]
    n_pad = -(-n_nodes // 128) * 128

    xa = jnp.zeros((n_pad, DHALF), jnp.float32).at[:n_nodes].set(
        node_features[:, :DHALF])
    xb = jnp.zeros((n_pad, DHALF), jnp.float32)
    xb = xb.at[:n_nodes, :d - DHALF].set(node_features[:, DHALF:])
    xb = xb.at[:n_nodes, 2].set(1.0)

    def wsplit(w):
        w16 = jnp.zeros((d, DPAD), jnp.float32).at[:, :d].set(w)
        wa = w16[:DHALF]
        wb = jnp.zeros((DHALF, DPAD), jnp.float32).at[:d - DHALF].set(w16[DHALF:])
        return wa, wb

    wsa1, wsb1 = wsplit(W_self1)
    wna1, wnb1 = wsplit(W_neigh1)
    wsa2, wsb2 = wsplit(W_self2)
    wna2, wnb2 = wsplit(W_neigh2)
    bp1 = jnp.zeros((DPAD,), jnp.float32).at[:d].set(b1).at[d].set(1.0)
    bp2 = jnp.zeros((DPAD,), jnp.float32).at[:d].set(b2)
    badd1 = jnp.zeros((8, DPAD), jnp.float32).at[0].set(bp1)
    badd2 = jnp.zeros((8, DPAD), jnp.float32).at[0].set(bp2)
    zer = jnp.zeros((ZCH, DHALF), jnp.float32)

    agg = _make_agg(n_pad, n_edges)
    combine1 = _make_combine(n_pad, split_out=True)
    combine2 = _make_combine(n_pad, split_out=False)
    score = _make_score(n_pad, n_edges, d)

    psrc, pdst = pos_edge_index[0], pos_edge_index[1]
    nsrc, ndst = neg_edge_index[0], neg_edge_index[1]

    g1 = agg(xa, xb, psrc, pdst, zer)
    h1a, h1b = combine1(xa, xb, g1, wsa1, wsb1, wna1, wnb1, badd1)
    g2 = agg(h1a, h1b, psrc, pdst, zer)
    h2 = combine2(h1a, h1b, g2, wsa2, wsb2, wna2, wnb2, badd2)
    pos, neg = score(h2, psrc, pdst, nsrc, ndst)
    return pos.reshape(n_edges, 1), neg.reshape(n_edges, 1)
